# baseline (speedup 1.0000x reference)
import functools

import jax
import jax.numpy as jnp
from jax import lax
from jax.experimental import pallas as pl
from jax.experimental.pallas import tpu as pltpu
from jax.experimental.pallas import tpu_sc as plsc


def _sc_gather_tail(emb, y, H):
    B = y.shape[0]
    _, D = emb.shape
    info = plsc.get_sparse_core_info()
    NC, NS = info.num_cores, info.num_subcores
    NW = NC * NS
    T = B - H
    bt = T // NW
    mesh = plsc.VectorSubcoreMesh(core_axis_name="c", subcore_axis_name="s")

    @functools.partial(
        pl.kernel,
        mesh=mesh,
        out_type=jax.ShapeDtypeStruct((T, D), jnp.float32),
        scratch_types=[
            pltpu.VMEM((bt,), jnp.int32),
            pltpu.VMEM((bt, D), jnp.float32),
            pltpu.SemaphoreType.DMA,
        ],
    )
    def gather_kernel(emb_hbm, y_hbm, out_hbm, idx_v, rows_v, sem):
        wid = lax.axis_index("s") * NC + lax.axis_index("c")
        base = wid * bt
        pltpu.sync_copy(y_hbm.at[pl.ds(H + base, bt)], idx_v)
        pltpu.async_copy(emb_hbm.at[idx_v], rows_v, sem).wait()
        pltpu.sync_copy(rows_v, out_hbm.at[pl.ds(base, bt)])

    return gather_kernel(emb, y)


def _head_body(nsteps, bb, y_sm, x_ref, emb_any, o_ref, ebuf, sem):
    i = pl.program_id(0)

    def issue(step, slot):
        for j in range(bb):
            pltpu.make_async_copy(
                emb_any.at[y_sm[step * bb + j]], ebuf.at[slot, j], sem.at[slot]
            ).start()

    def drain_and_add(step, slot):
        pltpu.make_async_copy(
            emb_any.at[pl.ds(0, bb)], ebuf.at[slot], sem.at[slot]
        ).wait()
        rows = ebuf[slot]
        o_ref[...] = x_ref[...] + rows[:, None, :]

    even = lax.rem(i, 2) == 0

    @pl.when(i == 0)
    def _():
        issue(0, 0)

    @pl.when(jnp.logical_and(i + 1 < nsteps, even))
    def _():
        issue(i + 1, 1)

    @pl.when(jnp.logical_and(i + 1 < nsteps, jnp.logical_not(even)))
    def _():
        issue(i + 1, 0)

    @pl.when(even)
    def _():
        drain_and_add(i, 0)

    @pl.when(jnp.logical_not(even))
    def _():
        drain_and_add(i, 1)


def _tc_head(x, y, emb, H, bb):
    B, S, D = x.shape
    nsteps = H // bb
    return pl.pallas_call(
        functools.partial(_head_body, nsteps, bb),
        grid_spec=pltpu.PrefetchScalarGridSpec(
            num_scalar_prefetch=1,
            grid=(nsteps,),
            in_specs=[
                pl.BlockSpec((bb, S, D), lambda i, yr: (i, 0, 0)),
                pl.BlockSpec(memory_space=pl.ANY),
            ],
            out_specs=pl.BlockSpec((bb, S, D), lambda i, yr: (i, 0, 0)),
            scratch_shapes=[
                pltpu.VMEM((2, bb, D), jnp.float32),
                pltpu.SemaphoreType.DMA((2,)),
            ],
        ),
        out_shape=jax.ShapeDtypeStruct((B, S, D), x.dtype),
    )(y, x, emb)


def _tail_body(x_ref, e_ref, _prev_ref, o_ref):
    o_ref[...] = x_ref[...] + e_ref[...][:, None, :]


def _tc_tail(x, e_tail, prev, H, bb):
    B, S, D = x.shape
    off = H // bb
    nsteps = (B - H) // bb
    return pl.pallas_call(
        _tail_body,
        grid=(nsteps,),
        in_specs=[
            pl.BlockSpec((bb, S, D), lambda i: (i + off, 0, 0)),
            pl.BlockSpec((bb, D), lambda i: (i, 0)),
            pl.BlockSpec(memory_space=pl.ANY),
        ],
        out_specs=pl.BlockSpec((bb, S, D), lambda i: (i + off, 0, 0)),
        out_shape=jax.ShapeDtypeStruct((B, S, D), x.dtype),
        input_output_aliases={2: 0},
    )(x, e_tail, prev)


def kernel(x, y, emb):
    y = y.astype(jnp.int32)
    H = 256
    e_tail = _sc_gather_tail(emb, y, H)
    out_head = _tc_head(x, y, emb, H, bb=32)
    return _tc_tail(x, e_tail, out_head, H, bb=128)


# speedup vs baseline: 1.0091x; 1.0091x over previous
import functools

import jax
import jax.numpy as jnp
from jax import lax
from jax.experimental import pallas as pl
from jax.experimental.pallas import tpu as pltpu
from jax.experimental.pallas import tpu_sc as plsc


def _sc_gather_tail(emb, y, H):
    B = y.shape[0]
    _, D = emb.shape
    info = plsc.get_sparse_core_info()
    NC, NS = info.num_cores, info.num_subcores
    NW = NC * NS
    T = B - H
    bt = T // NW
    mesh = plsc.VectorSubcoreMesh(core_axis_name="c", subcore_axis_name="s")

    @functools.partial(
        pl.kernel,
        mesh=mesh,
        out_type=jax.ShapeDtypeStruct((T, D), jnp.float32),
        scratch_types=[
            pltpu.VMEM((bt,), jnp.int32),
            pltpu.VMEM((bt, D), jnp.float32),
            pltpu.SemaphoreType.DMA,
        ],
    )
    def gather_kernel(emb_hbm, y_hbm, out_hbm, idx_v, rows_v, sem):
        wid = lax.axis_index("s") * NC + lax.axis_index("c")
        base = wid * bt
        pltpu.sync_copy(y_hbm.at[pl.ds(H + base, bt)], idx_v)
        pltpu.async_copy(emb_hbm.at[idx_v], rows_v, sem).wait()
        pltpu.sync_copy(rows_v, out_hbm.at[pl.ds(base, bt)])

    return gather_kernel(emb, y)


def _head_body(nsteps, bb, y_sm, x_ref, emb_any, o_ref, ebuf, sem):
    i = pl.program_id(0)

    def issue(step, slot):
        for j in range(bb):
            pltpu.make_async_copy(
                emb_any.at[y_sm[step * bb + j]], ebuf.at[slot, j], sem.at[slot]
            ).start()

    def drain_and_add(step, slot):
        pltpu.make_async_copy(
            emb_any.at[pl.ds(0, bb)], ebuf.at[slot], sem.at[slot]
        ).wait()
        rows = ebuf[slot]
        o_ref[...] = x_ref[...] + rows[:, None, :]

    even = lax.rem(i, 2) == 0

    @pl.when(i == 0)
    def _():
        issue(0, 0)

    @pl.when(jnp.logical_and(i + 1 < nsteps, even))
    def _():
        issue(i + 1, 1)

    @pl.when(jnp.logical_and(i + 1 < nsteps, jnp.logical_not(even)))
    def _():
        issue(i + 1, 0)

    @pl.when(even)
    def _():
        drain_and_add(i, 0)

    @pl.when(jnp.logical_not(even))
    def _():
        drain_and_add(i, 1)


def _tc_head(x, y, emb, H, bb):
    B, S, D = x.shape
    nsteps = H // bb
    return pl.pallas_call(
        functools.partial(_head_body, nsteps, bb),
        grid_spec=pltpu.PrefetchScalarGridSpec(
            num_scalar_prefetch=1,
            grid=(nsteps,),
            in_specs=[
                pl.BlockSpec((bb, S, D), lambda i, yr: (i, 0, 0)),
                pl.BlockSpec(memory_space=pl.ANY),
            ],
            out_specs=pl.BlockSpec((bb, S, D), lambda i, yr: (i, 0, 0)),
            scratch_shapes=[
                pltpu.VMEM((2, bb, D), jnp.float32),
                pltpu.SemaphoreType.DMA((2,)),
            ],
        ),
        out_shape=jax.ShapeDtypeStruct((B, S, D), x.dtype),
    )(y, x, emb)


def _tail_body(x_ref, e_ref, _prev_ref, o_ref):
    o_ref[...] = x_ref[...] + e_ref[...][:, None, :]


def _tc_tail(x, e_tail, prev, H, bb):
    B, S, D = x.shape
    off = H // bb
    nsteps = (B - H) // bb
    return pl.pallas_call(
        _tail_body,
        grid=(nsteps,),
        in_specs=[
            pl.BlockSpec((bb, S, D), lambda i: (i + off, 0, 0)),
            pl.BlockSpec((bb, D), lambda i: (i, 0)),
            pl.BlockSpec(memory_space=pl.ANY),
        ],
        out_specs=pl.BlockSpec((bb, S, D), lambda i: (i + off, 0, 0)),
        out_shape=jax.ShapeDtypeStruct((B, S, D), x.dtype),
        input_output_aliases={2: 0},
    )(x, e_tail, prev)


def kernel(x, y, emb):
    y = y.astype(jnp.int32)
    H = 256
    e_tail = _sc_gather_tail(emb, y, H)
    out_head = _tc_head(x, y, emb, H, bb=64)
    return _tc_tail(x, e_tail, out_head, H, bb=128)


# speedup vs baseline: 1.0105x; 1.0013x over previous
import functools

import jax
import jax.numpy as jnp
from jax import lax
from jax.experimental import pallas as pl
from jax.experimental.pallas import tpu as pltpu
from jax.experimental.pallas import tpu_sc as plsc


def _sc_gather_tail(emb, y, H):
    B = y.shape[0]
    _, D = emb.shape
    info = plsc.get_sparse_core_info()
    NC, NS = info.num_cores, info.num_subcores
    NW = NC * NS
    T = B - H
    bt = T // NW
    mesh = plsc.VectorSubcoreMesh(core_axis_name="c", subcore_axis_name="s")

    @functools.partial(
        pl.kernel,
        mesh=mesh,
        out_type=jax.ShapeDtypeStruct((T, D), jnp.float32),
        scratch_types=[
            pltpu.VMEM((bt,), jnp.int32),
            pltpu.VMEM((bt, D), jnp.float32),
            pltpu.SemaphoreType.DMA,
        ],
    )
    def gather_kernel(emb_hbm, y_hbm, out_hbm, idx_v, rows_v, sem):
        wid = lax.axis_index("s") * NC + lax.axis_index("c")
        base = wid * bt
        pltpu.sync_copy(y_hbm.at[pl.ds(H + base, bt)], idx_v)
        pltpu.async_copy(emb_hbm.at[idx_v], rows_v, sem).wait()
        pltpu.sync_copy(rows_v, out_hbm.at[pl.ds(base, bt)])

    return gather_kernel(emb, y)


def _head_body(nsteps, bb, y_sm, x_ref, emb_any, o_ref, ebuf, sem):
    i = pl.program_id(0)

    def issue(step, slot):
        for j in range(bb):
            pltpu.make_async_copy(
                emb_any.at[y_sm[step * bb + j]], ebuf.at[slot, j], sem.at[slot]
            ).start()

    def drain_and_add(step, slot):
        pltpu.make_async_copy(
            emb_any.at[pl.ds(0, bb)], ebuf.at[slot], sem.at[slot]
        ).wait()
        rows = ebuf[slot]
        o_ref[...] = x_ref[...] + rows[:, None, :]

    even = lax.rem(i, 2) == 0

    @pl.when(i == 0)
    def _():
        issue(0, 0)

    @pl.when(jnp.logical_and(i + 1 < nsteps, even))
    def _():
        issue(i + 1, 1)

    @pl.when(jnp.logical_and(i + 1 < nsteps, jnp.logical_not(even)))
    def _():
        issue(i + 1, 0)

    @pl.when(even)
    def _():
        drain_and_add(i, 0)

    @pl.when(jnp.logical_not(even))
    def _():
        drain_and_add(i, 1)


def _tc_head(x, y, emb, H, bb):
    B, S, D = x.shape
    nsteps = H // bb
    return pl.pallas_call(
        functools.partial(_head_body, nsteps, bb),
        grid_spec=pltpu.PrefetchScalarGridSpec(
            num_scalar_prefetch=1,
            grid=(nsteps,),
            in_specs=[
                pl.BlockSpec((bb, S, D), lambda i, yr: (i, 0, 0)),
                pl.BlockSpec(memory_space=pl.ANY),
            ],
            out_specs=pl.BlockSpec((bb, S, D), lambda i, yr: (i, 0, 0)),
            scratch_shapes=[
                pltpu.VMEM((2, bb, D), jnp.float32),
                pltpu.SemaphoreType.DMA((2,)),
            ],
        ),
        out_shape=jax.ShapeDtypeStruct((B, S, D), x.dtype),
    )(y, x, emb)


def _tail_body(x_ref, e_ref, _prev_ref, o_ref):
    o_ref[...] = x_ref[...] + e_ref[...][:, None, :]


def _tc_tail(x, e_tail, prev, H, bb):
    B, S, D = x.shape
    off = H // bb
    nsteps = (B - H) // bb
    return pl.pallas_call(
        _tail_body,
        grid=(nsteps,),
        in_specs=[
            pl.BlockSpec((bb, S, D), lambda i: (i + off, 0, 0)),
            pl.BlockSpec((bb, D), lambda i: (i, 0)),
            pl.BlockSpec(memory_space=pl.ANY),
        ],
        out_specs=pl.BlockSpec((bb, S, D), lambda i: (i + off, 0, 0)),
        out_shape=jax.ShapeDtypeStruct((B, S, D), x.dtype),
        input_output_aliases={2: 0},
    )(x, e_tail, prev)


def kernel(x, y, emb):
    y = y.astype(jnp.int32)
    H = 256
    out_head = _tc_head(x, y, emb, H, bb=64)
    e_tail = _sc_gather_tail(emb, y, H)
    return _tc_tail(x, e_tail, out_head, H, bb=128)
